# baseline (speedup 1.0000x reference)
import functools

import jax
import jax.numpy as jnp
from jax import lax
from jax.experimental import pallas as pl
from jax.experimental.pallas import tpu as pltpu
from jax.experimental.pallas import tpu_sc as plsc

_GAMMA = 12.0
_L = 16

_SIN_C = (
    0.9999999562127889,
    -0.16666631913872146,
    0.00833289061409179,
    -0.00019820756363012935,
    2.712799827662477e-06,
    -2.0872664575493573e-08,
)
_COS_C = (
    0.9999999922898474,
    -0.49999991770959235,
    0.04166652433757078,
    -0.0013887970265659048,
    2.4773420813397368e-05,
    -2.711333772339074e-07,
    1.7368996050969864e-09,
)


def _sincos(x):
    u = x * x
    s = jnp.float32(_SIN_C[5])
    for c in _SIN_C[4::-1]:
        s = s * u + jnp.float32(c)
    s = s * x
    c = jnp.float32(_COS_C[6])
    for cc in _COS_C[5::-1]:
        c = c * u + jnp.float32(cc)
    return s, c


def _sqrt(x):
    i = lax.bitcast_convert_type(x, jnp.int32)
    i = jnp.int32(0x5F3759DF) - lax.shift_right_arithmetic(i, 1)
    y = lax.bitcast_convert_type(i, jnp.float32)
    for _ in range(3):
        t = jnp.float32(0.5) * x * y
        y = y * (jnp.float32(1.5) - t * y)
    return x * y


def _compiler_params():
    return pltpu.CompilerParams(
        needs_layout_passes=False, use_tc_tiling_on_sc=True)


def _make_relayout_kernel(v, ncols, tail, g_rows):
    mesh = plsc.VectorSubcoreMesh(core_axis_name="c", subcore_axis_name="s")
    nc = mesh.num_cores
    nw = mesh.num_cores * mesh.num_subcores
    iters = (ncols + nw - 1) // nw
    g_struct = jax.ShapeDtypeStruct((g_rows, 128), jnp.float32)

    @functools.partial(
        pl.kernel,
        out_type=(g_struct, g_struct, g_struct),
        mesh=mesh,
        compiler_params=_compiler_params(),
        scratch_types=[
            pltpu.VMEM((32, 128), jnp.float32),
            pltpu.VMEM((32, 128), jnp.float32),
            pltpu.VMEM((32, 128), jnp.float32),
            pltpu.VMEM((32, 128), jnp.float32),
            pltpu.VMEM((64, 128), jnp.float32),
            pltpu.VMEM((16, 128), jnp.float32),
            pltpu.SemaphoreType.DMA,
            pltpu.SemaphoreType.DMA,
        ],
    )
    def relayout(e_t, ei_t, r_t, tails, g0, g1, g2,
                 s0, s1, p0, p1, tv, tp, sem_in, sem_out):
        wid = lax.axis_index("s") * nc + lax.axis_index("c")
        iota = lax.iota(jnp.int32, _L)
        sbufs = (s0, s1)
        pbufs = (p0, p1)

        def transpose_block(src, dst, n_rows):
            for gr in range(n_rows):
                for h in range(8):
                    j16 = (h % 2) * _L + iota
                    col = jnp.full((_L,), 4 * gr + h // 2, jnp.int32)
                    dst[gr, pl.ds(h * _L, _L)] = plsc.load_gather(
                        src, [j16, col])

        for tab, g in ((e_t, g0), (ei_t, g1), (r_t, g2)):
            pltpu.async_copy(
                tab.at[:, pl.ds(wid * 128, 128)], s0, sem_in)

            def body(t, carry, tab=tab, g=g):
                u = wid + nw * t
                for par in (0, 1):
                    @pl.when((t % 2 == par) & (u < ncols))
                    def _(par=par, u=u, tab=tab, g=g):
                        nxt = u + nw
                        @pl.when(nxt < ncols)
                        def _():
                            pltpu.async_copy(
                                tab.at[:, pl.ds(nxt * 128, 128)],
                                sbufs[1 - par], sem_in)
                        pltpu.make_async_copy(
                            tab.at[:, pl.ds(u * 128, 128)],
                            sbufs[par], sem_in).wait()
                        @pl.when(t >= 2)
                        def _():
                            pltpu.make_async_copy(
                                pbufs[par], g.at[pl.ds(0, 32)],
                                sem_out).wait()
                        transpose_block(sbufs[par], pbufs[par], 32)
                        pltpu.async_copy(
                            pbufs[par], g.at[pl.ds(u * 32, 32)], sem_out)
                return carry

            lax.fori_loop(0, iters, body, 0)
            for _i in range(2):
                pltpu.make_async_copy(
                    p0, g.at[pl.ds(0, 32)], sem_out).wait()

        if tail:
            for tb, g in ((0, g0), (1, g1), (2, g2)):
                @pl.when(wid == tb)
                def _(tb=tb, g=g):
                    pltpu.sync_copy(tails.at[pl.ds(64 * tb, 64)], tv)
                    for gr in range(16):
                        for h in range(8):
                            j16 = (h % 2) * _L + iota
                            row = jnp.full((_L,), 4 * gr + h // 2, jnp.int32)
                            tp[gr, pl.ds(h * _L, _L)] = plsc.load_gather(
                                tv, [row, j16])
                    pltpu.sync_copy(tp, g.at[pl.ds(ncols * 32, 16)])

    return relayout


def _make_score_kernel(batch, g_rows, bpw, chunk):
    mesh = plsc.VectorSubcoreMesh(core_axis_name="c", subcore_axis_name="s")
    nc = mesh.num_cores
    n_chunks = bpw // chunk

    @functools.partial(
        pl.kernel,
        out_type=jax.ShapeDtypeStruct((batch,), jnp.float32),
        mesh=mesh,
        compiler_params=_compiler_params(),
        scratch_types=[
            pltpu.VMEM((bpw,), jnp.int32),
            pltpu.VMEM((bpw,), jnp.int32),
            pltpu.VMEM((bpw,), jnp.int32),
            pltpu.VMEM((bpw,), jnp.int32),
            pltpu.VMEM((bpw,), jnp.int32),
            pltpu.VMEM((bpw,), jnp.int32),
            pltpu.VMEM((chunk, 128), jnp.float32),
            pltpu.VMEM((chunk, 128), jnp.float32),
            pltpu.VMEM((chunk, 128), jnp.float32),
            pltpu.VMEM((chunk, 128), jnp.float32),
            pltpu.VMEM((chunk, 128), jnp.float32),
            pltpu.VMEM((_L * bpw,), jnp.float32),
            pltpu.VMEM((bpw,), jnp.float32),
            pltpu.SemaphoreType.DMA,
        ],
    )
    def score_kernel(h_idx, r_idx, t_idx, g_ent, g_ent_im, g_rel, out,
                     ih, ir, it, qh, qr, qt,
                     bhe, bhi, brl, bte, bti, tr, sc, sem):
        wid = lax.axis_index("s") * nc + lax.axis_index("c")
        base = wid * bpw
        iota = lax.iota(jnp.int32, _L)

        pltpu.sync_copy(h_idx.at[pl.ds(base, bpw)], ih)
        pltpu.sync_copy(r_idx.at[pl.ds(base, bpw)], ir)
        pltpu.sync_copy(t_idx.at[pl.ds(base, bpw)], it)

        def quarter(g, carry):
            d = pl.ds(g * _L, _L)
            qh[d] = lax.shift_right_logical(ih[d], 2)
            qr[d] = lax.shift_right_logical(ir[d], 2)
            qt[d] = lax.shift_right_logical(it[d], 2)
            return carry

        lax.fori_loop(0, bpw // _L, quarter, 0)

        def chunk_body(ch, carry):
            cd = pl.ds(ch * chunk, chunk)
            cps = [
                pltpu.async_copy(g_ent.at[qh.at[cd]], bhe, sem),
                pltpu.async_copy(g_ent_im.at[qh.at[cd]], bhi, sem),
                pltpu.async_copy(g_rel.at[qr.at[cd]], brl, sem),
                pltpu.async_copy(g_ent.at[qt.at[cd]], bte, sem),
                pltpu.async_copy(g_ent_im.at[qt.at[cd]], bti, sem),
            ]
            for cp in cps:
                cp.wait()

            def grp(gq, c2):
                s16 = pl.ds(ch * chunk + gq * _L, _L)
                vh = ih[s16]
                vr = ir[s16]
                vt = it[s16]
                for lane in range(_L):
                    row = gq * _L + lane
                    oh = (vh[lane] & 3) * 32
                    orr = (vr[lane] & 3) * 32
                    ot = (vt[lane] & 3) * 32
                    acc = None
                    for half in range(2):
                        ho = half * _L
                        rl = brl[row, pl.ds(orr + ho, _L)]
                        sn, cs = _sincos(rl)
                        he = bhe[row, pl.ds(oh + ho, _L)]
                        hi = bhi[row, pl.ds(oh + ho, _L)]
                        te = bte[row, pl.ds(ot + ho, _L)]
                        ti = bti[row, pl.ds(ot + ho, _L)]
                        s_re = he * cs - hi * sn - te
                        s_im = he * sn + hi * cs - ti
                        m = _sqrt(s_re * s_re + s_im * s_im)
                        acc = m if acc is None else acc + m
                    plsc.store_scatter(
                        tr,
                        [iota * bpw + (ch * chunk + gq * _L + lane)],
                        acc)
                return c2

            lax.fori_loop(0, chunk // _L, grp, 0)
            return carry

        lax.fori_loop(0, n_chunks, chunk_body, 0)

        def rowsum(g, carry):
            acc = tr[pl.ds(g * _L, _L)]
            for k in range(1, _L):
                acc = acc + tr[pl.ds(k * bpw + g * _L, _L)]
            sc[pl.ds(g * _L, _L)] = acc - jnp.float32(_GAMMA)
            return carry

        lax.fori_loop(0, bpw // _L, rowsum, 0)
        pltpu.sync_copy(sc, out.at[pl.ds(base, bpw)])

    return score_kernel


def kernel(pos_sample, ent_embd, ent_embd_im, rel_embd):
    batch = pos_sample.shape[0]
    v = ent_embd.shape[0]
    ncols = v // 128
    tail = v - ncols * 128
    g_rows = (v + 3) // 4
    n_workers = 32
    bpw = batch // n_workers

    e_t = ent_embd.T
    ei_t = ent_embd_im.T
    r_t = rel_embd.T
    tails = jnp.concatenate([
        jnp.pad(t[ncols * 128:], ((0, 64 - tail), (0, 128 - t.shape[1])))
        for t in (ent_embd, ent_embd_im, rel_embd)
    ], axis=0)

    relayout = _make_relayout_kernel(v, ncols, tail, g_rows)
    g_ent, g_ent_im, g_rel = relayout(e_t, ei_t, r_t, tails)

    score_k = _make_score_kernel(batch, g_rows, bpw, 128)
    score = score_k(pos_sample[:, 0], pos_sample[:, 1], pos_sample[:, 2],
                    g_ent, g_ent_im, g_rel)
    return score.reshape(batch, 1)


# speedup vs baseline: 1.4630x; 1.4630x over previous
import functools

import jax
import jax.numpy as jnp
from jax import lax
from jax.experimental import pallas as pl
from jax.experimental.pallas import tpu as pltpu
from jax.experimental.pallas import tpu_sc as plsc

_GAMMA = 12.0
_L = 16

_SIN_C = (
    0.9999999562127889,
    -0.16666631913872146,
    0.00833289061409179,
    -0.00019820756363012935,
    2.712799827662477e-06,
    -2.0872664575493573e-08,
)
_COS_C = (
    0.9999999922898474,
    -0.49999991770959235,
    0.04166652433757078,
    -0.0013887970265659048,
    2.4773420813397368e-05,
    -2.711333772339074e-07,
    1.7368996050969864e-09,
)


def _sincos(x):
    u = x * x
    s = jnp.float32(_SIN_C[5])
    for c in _SIN_C[4::-1]:
        s = s * u + jnp.float32(c)
    s = s * x
    c = jnp.float32(_COS_C[6])
    for cc in _COS_C[5::-1]:
        c = c * u + jnp.float32(cc)
    return s, c


def _sqrt(x):
    i = lax.bitcast_convert_type(x, jnp.int32)
    i = jnp.int32(0x5F3759DF) - lax.shift_right_arithmetic(i, 1)
    y = lax.bitcast_convert_type(i, jnp.float32)
    for _ in range(3):
        t = jnp.float32(0.5) * x * y
        y = y * (jnp.float32(1.5) - t * y)
    return x * y


def _compiler_params():
    return pltpu.CompilerParams(
        needs_layout_passes=False, use_tc_tiling_on_sc=True)


def _make_relayout_kernel(v, ncols, tail, g_rows):
    mesh = plsc.VectorSubcoreMesh(core_axis_name="c", subcore_axis_name="s")
    nc = mesh.num_cores
    nw = mesh.num_cores * mesh.num_subcores
    iters = (ncols + nw - 1) // nw
    g_struct = jax.ShapeDtypeStruct((g_rows, 128), jnp.float32)

    @functools.partial(
        pl.kernel,
        out_type=(g_struct, g_struct, g_struct),
        mesh=mesh,
        compiler_params=_compiler_params(),
        scratch_types=[
            pltpu.VMEM((32, 128), jnp.float32),
            pltpu.VMEM((32, 128), jnp.float32),
            pltpu.VMEM((32, 128), jnp.float32),
            pltpu.VMEM((32, 128), jnp.float32),
            pltpu.VMEM((64, 128), jnp.float32),
            pltpu.VMEM((16, 128), jnp.float32),
            pltpu.SemaphoreType.DMA,
            pltpu.SemaphoreType.DMA,
        ],
    )
    def relayout(e_t, ei_t, r_t, tails, g0, g1, g2,
                 s0, s1, p0, p1, tv, tp, sem_in, sem_out):
        wid = lax.axis_index("s") * nc + lax.axis_index("c")
        iota = lax.iota(jnp.int32, _L)
        sbufs = (s0, s1)
        pbufs = (p0, p1)

        def transpose_block(src, dst, n_rows):
            for gr0 in range(0, n_rows, 4):
                vals = []
                for gr in range(gr0, gr0 + 4):
                    for h in range(8):
                        j16 = (h % 2) * _L + iota
                        col = jnp.full((_L,), 4 * gr + h // 2, jnp.int32)
                        vals.append(plsc.load_gather(src, [j16, col]))
                for q, gr in enumerate(range(gr0, gr0 + 4)):
                    for h in range(8):
                        dst[gr, pl.ds(h * _L, _L)] = vals[q * 8 + h]

        for tab, g in ((e_t, g0), (ei_t, g1), (r_t, g2)):
            pltpu.async_copy(
                tab.at[:, pl.ds(wid * 128, 128)], s0, sem_in)

            def body(t, carry, tab=tab, g=g):
                u = wid + nw * t
                for par in (0, 1):
                    @pl.when((t % 2 == par) & (u < ncols))
                    def _(par=par, u=u, tab=tab, g=g):
                        nxt = u + nw
                        @pl.when(nxt < ncols)
                        def _():
                            pltpu.async_copy(
                                tab.at[:, pl.ds(nxt * 128, 128)],
                                sbufs[1 - par], sem_in)
                        pltpu.make_async_copy(
                            tab.at[:, pl.ds(u * 128, 128)],
                            sbufs[par], sem_in).wait()
                        @pl.when(t >= 2)
                        def _():
                            pltpu.make_async_copy(
                                pbufs[par], g.at[pl.ds(0, 32)],
                                sem_out).wait()
                        transpose_block(sbufs[par], pbufs[par], 32)
                        pltpu.async_copy(
                            pbufs[par], g.at[pl.ds(u * 32, 32)], sem_out)
                return carry

            lax.fori_loop(0, iters, body, 0)
            for _i in range(2):
                pltpu.make_async_copy(
                    p0, g.at[pl.ds(0, 32)], sem_out).wait()

        if tail:
            for tb, g in ((0, g0), (1, g1), (2, g2)):
                @pl.when(wid == tb)
                def _(tb=tb, g=g):
                    pltpu.sync_copy(tails.at[pl.ds(64 * tb, 64)], tv)
                    tvals = []
                    for gr in range(16):
                        for h in range(8):
                            j16 = (h % 2) * _L + iota
                            row = jnp.full((_L,), 4 * gr + h // 2, jnp.int32)
                            tvals.append(plsc.load_gather(tv, [row, j16]))
                    for gr in range(16):
                        for h in range(8):
                            tp[gr, pl.ds(h * _L, _L)] = tvals[gr * 8 + h]
                    pltpu.sync_copy(tp, g.at[pl.ds(ncols * 32, 16)])

    return relayout


def _make_score_kernel(batch, g_rows, bpw, chunk):
    mesh = plsc.VectorSubcoreMesh(core_axis_name="c", subcore_axis_name="s")
    nc = mesh.num_cores
    n_chunks = bpw // chunk

    @functools.partial(
        pl.kernel,
        out_type=jax.ShapeDtypeStruct((batch,), jnp.float32),
        mesh=mesh,
        compiler_params=_compiler_params(),
        scratch_types=[
            pltpu.VMEM((bpw,), jnp.int32),
            pltpu.VMEM((bpw,), jnp.int32),
            pltpu.VMEM((bpw,), jnp.int32),
            pltpu.VMEM((bpw,), jnp.int32),
            pltpu.VMEM((bpw,), jnp.int32),
            pltpu.VMEM((bpw,), jnp.int32),
            pltpu.VMEM((chunk, 128), jnp.float32),
            pltpu.VMEM((chunk, 128), jnp.float32),
            pltpu.VMEM((chunk, 128), jnp.float32),
            pltpu.VMEM((chunk, 128), jnp.float32),
            pltpu.VMEM((chunk, 128), jnp.float32),
            pltpu.VMEM((_L * bpw,), jnp.float32),
            pltpu.VMEM((bpw,), jnp.float32),
            pltpu.SemaphoreType.DMA,
        ],
    )
    def score_kernel(h_idx, r_idx, t_idx, g_ent, g_ent_im, g_rel, out,
                     ih, ir, it, qh, qr, qt,
                     bhe, bhi, brl, bte, bti, tr, sc, sem):
        wid = lax.axis_index("s") * nc + lax.axis_index("c")
        base = wid * bpw
        iota = lax.iota(jnp.int32, _L)

        pltpu.sync_copy(h_idx.at[pl.ds(base, bpw)], ih)
        pltpu.sync_copy(r_idx.at[pl.ds(base, bpw)], ir)
        pltpu.sync_copy(t_idx.at[pl.ds(base, bpw)], it)

        def quarter(g, carry):
            d = pl.ds(g * _L, _L)
            qh[d] = lax.shift_right_logical(ih[d], 2)
            qr[d] = lax.shift_right_logical(ir[d], 2)
            qt[d] = lax.shift_right_logical(it[d], 2)
            return carry

        lax.fori_loop(0, bpw // _L, quarter, 0)

        def chunk_body(ch, carry):
            cd = pl.ds(ch * chunk, chunk)
            cps = [
                pltpu.async_copy(g_ent.at[qh.at[cd]], bhe, sem),
                pltpu.async_copy(g_ent_im.at[qh.at[cd]], bhi, sem),
                pltpu.async_copy(g_rel.at[qr.at[cd]], brl, sem),
                pltpu.async_copy(g_ent.at[qt.at[cd]], bte, sem),
                pltpu.async_copy(g_ent_im.at[qt.at[cd]], bti, sem),
            ]
            for cp in cps:
                cp.wait()

            def grp(gq, c2):
                s16 = pl.ds(ch * chunk + gq * _L, _L)
                vh = ih[s16]
                vr = ir[s16]
                vt = it[s16]
                for lane in range(_L):
                    row = gq * _L + lane
                    oh = (vh[lane] & 3) * 32
                    orr = (vr[lane] & 3) * 32
                    ot = (vt[lane] & 3) * 32
                    acc = None
                    for half in range(2):
                        ho = half * _L
                        rl = brl[row, pl.ds(orr + ho, _L)]
                        sn, cs = _sincos(rl)
                        he = bhe[row, pl.ds(oh + ho, _L)]
                        hi = bhi[row, pl.ds(oh + ho, _L)]
                        te = bte[row, pl.ds(ot + ho, _L)]
                        ti = bti[row, pl.ds(ot + ho, _L)]
                        s_re = he * cs - hi * sn - te
                        s_im = he * sn + hi * cs - ti
                        m = _sqrt(s_re * s_re + s_im * s_im)
                        acc = m if acc is None else acc + m
                    plsc.store_scatter(
                        tr,
                        [iota * bpw + (ch * chunk + gq * _L + lane)],
                        acc)
                return c2

            lax.fori_loop(0, chunk // _L, grp, 0)
            return carry

        lax.fori_loop(0, n_chunks, chunk_body, 0)

        def rowsum(g, carry):
            acc = tr[pl.ds(g * _L, _L)]
            for k in range(1, _L):
                acc = acc + tr[pl.ds(k * bpw + g * _L, _L)]
            sc[pl.ds(g * _L, _L)] = acc - jnp.float32(_GAMMA)
            return carry

        lax.fori_loop(0, bpw // _L, rowsum, 0)
        pltpu.sync_copy(sc, out.at[pl.ds(base, bpw)])

    return score_kernel


def kernel(pos_sample, ent_embd, ent_embd_im, rel_embd):
    batch = pos_sample.shape[0]
    v = ent_embd.shape[0]
    ncols = v // 128
    tail = v - ncols * 128
    g_rows = (v + 3) // 4
    n_workers = 32
    bpw = batch // n_workers

    e_t = ent_embd.T
    ei_t = ent_embd_im.T
    r_t = rel_embd.T
    tails = jnp.concatenate([
        jnp.pad(t[ncols * 128:], ((0, 64 - tail), (0, 128 - t.shape[1])))
        for t in (ent_embd, ent_embd_im, rel_embd)
    ], axis=0)

    relayout = _make_relayout_kernel(v, ncols, tail, g_rows)
    g_ent, g_ent_im, g_rel = relayout(e_t, ei_t, r_t, tails)

    score_k = _make_score_kernel(batch, g_rows, bpw, 128)
    score = score_k(pos_sample[:, 0], pos_sample[:, 1], pos_sample[:, 2],
                    g_ent, g_ent_im, g_rel)
    return score.reshape(batch, 1)


# speedup vs baseline: 1.5902x; 1.0870x over previous
import functools

import jax
import jax.numpy as jnp
import numpy as np
from jax import lax
from jax.experimental import pallas as pl
from jax.experimental.pallas import tpu as pltpu
from jax.experimental.pallas import tpu_sc as plsc

_GAMMA = 12.0
_L = 16

_SIN_C = (
    0.9999999562127889,
    -0.16666631913872146,
    0.00833289061409179,
    -0.00019820756363012935,
    2.712799827662477e-06,
    -2.0872664575493573e-08,
)
_COS_C = (
    0.9999999922898474,
    -0.49999991770959235,
    0.04166652433757078,
    -0.0013887970265659048,
    2.4773420813397368e-05,
    -2.711333772339074e-07,
    1.7368996050969864e-09,
)


def _sincos(x):
    u = x * x
    s = jnp.float32(_SIN_C[5])
    for c in _SIN_C[4::-1]:
        s = s * u + jnp.float32(c)
    s = s * x
    c = jnp.float32(_COS_C[6])
    for cc in _COS_C[5::-1]:
        c = c * u + jnp.float32(cc)
    return s, c


def _sqrt(x):
    i = lax.bitcast_convert_type(x, jnp.int32)
    i = jnp.int32(0x5F3759DF) - lax.shift_right_arithmetic(i, 1)
    y = lax.bitcast_convert_type(i, jnp.float32)
    for _ in range(3):
        t = jnp.float32(0.5) * x * y
        y = y * (jnp.float32(1.5) - t * y)
    return x * y


def _compiler_params():
    return pltpu.CompilerParams(
        needs_layout_passes=False, use_tc_tiling_on_sc=True)


def _make_relayout_kernel(v, ncols, tail, g_rows):
    mesh = plsc.VectorSubcoreMesh(core_axis_name="c", subcore_axis_name="s")
    nc = mesh.num_cores
    nw = mesh.num_cores * mesh.num_subcores
    npair = ncols // 2
    iters = (npair + nw - 1) // nw
    g_struct = jax.ShapeDtypeStruct((g_rows, 128), jnp.float32)

    @functools.partial(
        pl.kernel,
        out_type=(g_struct, g_struct, g_struct),
        mesh=mesh,
        compiler_params=_compiler_params(),
        scratch_types=[
            pltpu.VMEM((32, 256), jnp.float32),
            pltpu.VMEM((32, 256), jnp.float32),
            pltpu.VMEM((64, 128), jnp.float32),
            pltpu.VMEM((64, 128), jnp.float32),
            pltpu.VMEM((64, 128), jnp.float32),
            pltpu.VMEM((16, 128), jnp.float32),
            pltpu.SemaphoreType.DMA,
            pltpu.SemaphoreType.DMA,
        ],
    )
    def relayout(e_t, ei_t, r_t, tails, g0, g1, g2,
                 s0, s1, p0, p1, tv, tp, sem_in, sem_out):
        wid = lax.axis_index("s") * nc + lax.axis_index("c")
        iota = lax.iota(jnp.int32, _L)
        sbufs = (s0, s1)
        pbufs = (p0, p1)

        row_base = [lax.shift_right_logical(iota + 16 * c16, 1 + 1)
                    for c16 in range(16)]
        col_base = [lax.shift_left((iota + 16 * c16) & 3, 5)
                    for c16 in range(16)]

        def transpose_pair(src, dst):
            for c16 in range(16):
                for j0 in range(0, 32, 8):
                    vals = [src[j, pl.ds(c16 * _L, _L)]
                            for j in range(j0, j0 + 8)]
                    for q, j in enumerate(range(j0, j0 + 8)):
                        plsc.store_scatter(
                            dst, [row_base[c16], col_base[c16] + j], vals[q])

        for tab, g in ((e_t, g0), (ei_t, g1), (r_t, g2)):
            pltpu.async_copy(
                tab.at[:, pl.ds(wid * 256, 256)], s0, sem_in)

            def body(t, carry, tab=tab, g=g):
                u = wid + nw * t
                for par in (0, 1):
                    @pl.when((t % 2 == par) & (u < npair))
                    def _(par=par, u=u, tab=tab, g=g):
                        nxt = u + nw
                        @pl.when(nxt < npair)
                        def _():
                            pltpu.async_copy(
                                tab.at[:, pl.ds(nxt * 256, 256)],
                                sbufs[1 - par], sem_in)
                        pltpu.make_async_copy(
                            tab.at[:, pl.ds(u * 256, 256)],
                            sbufs[par], sem_in).wait()
                        @pl.when(t >= 2)
                        def _():
                            pltpu.make_async_copy(
                                pbufs[par], g.at[pl.ds(0, 64)],
                                sem_out).wait()
                        transpose_pair(sbufs[par], pbufs[par])
                        pltpu.async_copy(
                            pbufs[par], g.at[pl.ds(u * 64, 64)], sem_out)
                return carry

            lax.fori_loop(0, iters, body, 0)
            for _i in range(2):
                pltpu.make_async_copy(
                    p0, g.at[pl.ds(0, 64)], sem_out).wait()

        if tail:
            for tb, g in ((0, g0), (1, g1), (2, g2)):
                @pl.when(wid == tb)
                def _(tb=tb, g=g):
                    pltpu.sync_copy(tails.at[pl.ds(64 * tb, 64)], tv)
                    tvals = []
                    for gr in range(16):
                        for h in range(8):
                            j16 = (h % 2) * _L + iota
                            row = jnp.full((_L,), 4 * gr + h // 2, jnp.int32)
                            tvals.append(plsc.load_gather(tv, [row, j16]))
                    for gr in range(16):
                        for h in range(8):
                            tp[gr, pl.ds(h * _L, _L)] = tvals[gr * 8 + h]
                    pltpu.sync_copy(tp, g.at[pl.ds(ncols * 32, 16)])

    return relayout


def _make_score_kernel(batch, g_rows, bpw, chunk):
    mesh = plsc.VectorSubcoreMesh(core_axis_name="c", subcore_axis_name="s")
    nc = mesh.num_cores
    n_chunks = bpw // chunk

    @functools.partial(
        pl.kernel,
        out_type=jax.ShapeDtypeStruct((batch,), jnp.float32),
        mesh=mesh,
        compiler_params=_compiler_params(),
        scratch_types=[
            pltpu.VMEM((bpw,), jnp.int32),
            pltpu.VMEM((bpw,), jnp.int32),
            pltpu.VMEM((bpw,), jnp.int32),
            pltpu.VMEM((bpw,), jnp.int32),
            pltpu.VMEM((bpw,), jnp.int32),
            pltpu.VMEM((bpw,), jnp.int32),
            pltpu.VMEM((chunk, 128), jnp.float32),
            pltpu.VMEM((chunk, 128), jnp.float32),
            pltpu.VMEM((chunk, 128), jnp.float32),
            pltpu.VMEM((chunk, 128), jnp.float32),
            pltpu.VMEM((chunk, 128), jnp.float32),
            pltpu.VMEM((_L * bpw,), jnp.float32),
            pltpu.VMEM((bpw,), jnp.float32),
            pltpu.SemaphoreType.DMA,
        ],
    )
    def score_kernel(h_idx, r_idx, t_idx, g_ent, g_ent_im, g_rel, out,
                     ih, ir, it, qh, qr, qt,
                     bhe, bhi, brl, bte, bti, tr, sc, sem):
        wid = lax.axis_index("s") * nc + lax.axis_index("c")
        base = wid * bpw
        iota = lax.iota(jnp.int32, _L)

        pltpu.sync_copy(h_idx.at[pl.ds(base, bpw)], ih)
        pltpu.sync_copy(r_idx.at[pl.ds(base, bpw)], ir)
        pltpu.sync_copy(t_idx.at[pl.ds(base, bpw)], it)

        def quarter(g, carry):
            d = pl.ds(g * _L, _L)
            qh[d] = lax.shift_right_logical(ih[d], 2)
            qr[d] = lax.shift_right_logical(ir[d], 2)
            qt[d] = lax.shift_right_logical(it[d], 2)
            return carry

        lax.fori_loop(0, bpw // _L, quarter, 0)

        def chunk_body(ch, carry):
            cd = pl.ds(ch * chunk, chunk)
            cps = [
                pltpu.async_copy(g_ent.at[qh.at[cd]], bhe, sem),
                pltpu.async_copy(g_ent_im.at[qh.at[cd]], bhi, sem),
                pltpu.async_copy(g_rel.at[qr.at[cd]], brl, sem),
                pltpu.async_copy(g_ent.at[qt.at[cd]], bte, sem),
                pltpu.async_copy(g_ent_im.at[qt.at[cd]], bti, sem),
            ]
            for cp in cps:
                cp.wait()

            def grp(gq, c2):
                s16 = pl.ds(ch * chunk + gq * _L, _L)
                vh = ih[s16]
                vr = ir[s16]
                vt = it[s16]
                for lane in range(_L):
                    row = gq * _L + lane
                    oh = (vh[lane] & 3) * 32
                    orr = (vr[lane] & 3) * 32
                    ot = (vt[lane] & 3) * 32
                    acc = None
                    for half in range(2):
                        ho = half * _L
                        rl = brl[row, pl.ds(orr + ho, _L)]
                        sn, cs = _sincos(rl)
                        he = bhe[row, pl.ds(oh + ho, _L)]
                        hi = bhi[row, pl.ds(oh + ho, _L)]
                        te = bte[row, pl.ds(ot + ho, _L)]
                        ti = bti[row, pl.ds(ot + ho, _L)]
                        s_re = he * cs - hi * sn - te
                        s_im = he * sn + hi * cs - ti
                        m = _sqrt(s_re * s_re + s_im * s_im)
                        acc = m if acc is None else acc + m
                    plsc.store_scatter(
                        tr,
                        [iota * bpw + (ch * chunk + gq * _L + lane)],
                        acc)
                return c2

            lax.fori_loop(0, chunk // _L, grp, 0)
            return carry

        lax.fori_loop(0, n_chunks, chunk_body, 0)

        def rowsum(g, carry):
            acc = tr[pl.ds(g * _L, _L)]
            for k in range(1, _L):
                acc = acc + tr[pl.ds(k * bpw + g * _L, _L)]
            sc[pl.ds(g * _L, _L)] = acc - jnp.float32(_GAMMA)
            return carry

        lax.fori_loop(0, bpw // _L, rowsum, 0)
        pltpu.sync_copy(sc, out.at[pl.ds(base, bpw)])

    return score_kernel


def kernel(pos_sample, ent_embd, ent_embd_im, rel_embd):
    batch = pos_sample.shape[0]
    v = ent_embd.shape[0]
    ncols = v // 128
    tail = v - ncols * 128
    g_rows = (v + 3) // 4
    n_workers = 32
    bpw = batch // n_workers

    e_t = ent_embd.T
    ei_t = ent_embd_im.T
    r_t = rel_embd.T
    tails = jnp.concatenate([
        jnp.pad(t[ncols * 128:], ((0, 64 - tail), (0, 128 - t.shape[1])))
        for t in (ent_embd, ent_embd_im, rel_embd)
    ], axis=0)

    relayout = _make_relayout_kernel(v, ncols, tail, g_rows)
    g_ent, g_ent_im, g_rel = relayout(e_t, ei_t, r_t, tails)

    score_k = _make_score_kernel(batch, g_rows, bpw, 128)
    score = score_k(pos_sample[:, 0], pos_sample[:, 1], pos_sample[:, 2],
                    g_ent, g_ent_im, g_rel)
    return score.reshape(batch, 1)


# speedup vs baseline: 6.5982x; 4.1493x over previous
import functools

import jax
import jax.numpy as jnp
import numpy as np
from jax import lax
from jax.experimental import pallas as pl
from jax.experimental.pallas import tpu as pltpu
from jax.experimental.pallas import tpu_sc as plsc

_GAMMA = 12.0
_L = 16

_SIN_C = (
    0.9999999562127889,
    -0.16666631913872146,
    0.00833289061409179,
    -0.00019820756363012935,
    2.712799827662477e-06,
    -2.0872664575493573e-08,
)
_COS_C = (
    0.9999999922898474,
    -0.49999991770959235,
    0.04166652433757078,
    -0.0013887970265659048,
    2.4773420813397368e-05,
    -2.711333772339074e-07,
    1.7368996050969864e-09,
)


def _sincos(x):
    u = x * x
    s = jnp.float32(_SIN_C[5])
    for c in _SIN_C[4::-1]:
        s = s * u + jnp.float32(c)
    s = s * x
    c = jnp.float32(_COS_C[6])
    for cc in _COS_C[5::-1]:
        c = c * u + jnp.float32(cc)
    return s, c


def _sqrt(x):
    i = lax.bitcast_convert_type(x, jnp.int32)
    i = jnp.int32(0x5F3759DF) - lax.shift_right_arithmetic(i, 1)
    y = lax.bitcast_convert_type(i, jnp.float32)
    for _ in range(3):
        t = jnp.float32(0.5) * x * y
        y = y * (jnp.float32(1.5) - t * y)
    return x * y


def _compiler_params():
    return pltpu.CompilerParams(
        needs_layout_passes=False, use_tc_tiling_on_sc=True)


def _make_relayout_kernel(v, ncols, tail, g_rows):
    mesh = plsc.VectorSubcoreMesh(core_axis_name="c", subcore_axis_name="s")
    nc = mesh.num_cores
    nw = mesh.num_cores * mesh.num_subcores
    npair = ncols // 2
    iters = (npair + nw - 1) // nw
    g_struct = jax.ShapeDtypeStruct((g_rows, 128), jnp.float32)

    @functools.partial(
        pl.kernel,
        out_type=(g_struct, g_struct, g_struct),
        mesh=mesh,
        compiler_params=_compiler_params(),
        scratch_types=[
            pltpu.VMEM((32, 256), jnp.float32),
            pltpu.VMEM((32, 256), jnp.float32),
            pltpu.VMEM((64, 128), jnp.float32),
            pltpu.VMEM((64, 128), jnp.float32),
            pltpu.VMEM((64, 128), jnp.float32),
            pltpu.VMEM((16, 128), jnp.float32),
            pltpu.SemaphoreType.DMA,
            pltpu.SemaphoreType.DMA,
        ],
    )
    def relayout(e_t, ei_t, r_t, tails, g0, g1, g2,
                 s0, s1, p0, p1, tv, tp, sem_in, sem_out):
        wid = lax.axis_index("s") * nc + lax.axis_index("c")
        iota = lax.iota(jnp.int32, _L)
        sbufs = (s0, s1)
        pbufs = (p0, p1)

        row_base = [lax.shift_right_logical(iota + 16 * c16, 1 + 1)
                    for c16 in range(16)]
        col_base = [lax.shift_left((iota + 16 * c16) & 3, 5)
                    for c16 in range(16)]

        def transpose_pair(src, dst):
            for c16 in range(16):
                for j0 in range(0, 32, 8):
                    vals = [src[j, pl.ds(c16 * _L, _L)]
                            for j in range(j0, j0 + 8)]
                    for q, j in enumerate(range(j0, j0 + 8)):
                        plsc.store_scatter(
                            dst, [row_base[c16], col_base[c16] + j], vals[q])

        for tab, g in ((e_t, g0), (ei_t, g1), (r_t, g2)):
            pltpu.async_copy(
                tab.at[:, pl.ds(wid * 256, 256)], s0, sem_in)

            def body(t, carry, tab=tab, g=g):
                u = wid + nw * t
                for par in (0, 1):
                    @pl.when((t % 2 == par) & (u < npair))
                    def _(par=par, u=u, tab=tab, g=g):
                        nxt = u + nw
                        @pl.when(nxt < npair)
                        def _():
                            pltpu.async_copy(
                                tab.at[:, pl.ds(nxt * 256, 256)],
                                sbufs[1 - par], sem_in)
                        pltpu.make_async_copy(
                            tab.at[:, pl.ds(u * 256, 256)],
                            sbufs[par], sem_in).wait()
                        @pl.when(t >= 2)
                        def _():
                            pltpu.make_async_copy(
                                pbufs[par], g.at[pl.ds(0, 64)],
                                sem_out).wait()
                        pltpu.async_copy(
                            pbufs[par], g.at[pl.ds(u * 64, 64)], sem_out)
                return carry

            lax.fori_loop(0, iters, body, 0)
            for _i in range(2):
                pltpu.make_async_copy(
                    p0, g.at[pl.ds(0, 64)], sem_out).wait()

        if tail:
            for tb, g in ((0, g0), (1, g1), (2, g2)):
                @pl.when(wid == tb)
                def _(tb=tb, g=g):
                    pltpu.sync_copy(tails.at[pl.ds(64 * tb, 64)], tv)
                    tvals = []
                    for gr in range(16):
                        for h in range(8):
                            j16 = (h % 2) * _L + iota
                            row = jnp.full((_L,), 4 * gr + h // 2, jnp.int32)
                            tvals.append(plsc.load_gather(tv, [row, j16]))
                    for gr in range(16):
                        for h in range(8):
                            tp[gr, pl.ds(h * _L, _L)] = tvals[gr * 8 + h]
                    pltpu.sync_copy(tp, g.at[pl.ds(ncols * 32, 16)])

    return relayout


def _make_score_kernel(batch, g_rows, bpw, chunk):
    mesh = plsc.VectorSubcoreMesh(core_axis_name="c", subcore_axis_name="s")
    nc = mesh.num_cores
    n_chunks = bpw // chunk

    @functools.partial(
        pl.kernel,
        out_type=jax.ShapeDtypeStruct((batch,), jnp.float32),
        mesh=mesh,
        compiler_params=_compiler_params(),
        scratch_types=[
            pltpu.VMEM((bpw,), jnp.int32),
            pltpu.VMEM((bpw,), jnp.int32),
            pltpu.VMEM((bpw,), jnp.int32),
            pltpu.VMEM((bpw,), jnp.int32),
            pltpu.VMEM((bpw,), jnp.int32),
            pltpu.VMEM((bpw,), jnp.int32),
            pltpu.VMEM((chunk, 128), jnp.float32),
            pltpu.VMEM((chunk, 128), jnp.float32),
            pltpu.VMEM((chunk, 128), jnp.float32),
            pltpu.VMEM((chunk, 128), jnp.float32),
            pltpu.VMEM((chunk, 128), jnp.float32),
            pltpu.VMEM((_L * bpw,), jnp.float32),
            pltpu.VMEM((bpw,), jnp.float32),
            pltpu.SemaphoreType.DMA,
        ],
    )
    def score_kernel(h_idx, r_idx, t_idx, g_ent, g_ent_im, g_rel, out,
                     ih, ir, it, qh, qr, qt,
                     bhe, bhi, brl, bte, bti, tr, sc, sem):
        wid = lax.axis_index("s") * nc + lax.axis_index("c")
        base = wid * bpw
        iota = lax.iota(jnp.int32, _L)

        pltpu.sync_copy(h_idx.at[pl.ds(base, bpw)], ih)
        pltpu.sync_copy(r_idx.at[pl.ds(base, bpw)], ir)
        pltpu.sync_copy(t_idx.at[pl.ds(base, bpw)], it)

        def quarter(g, carry):
            d = pl.ds(g * _L, _L)
            qh[d] = lax.shift_right_logical(ih[d], 2)
            qr[d] = lax.shift_right_logical(ir[d], 2)
            qt[d] = lax.shift_right_logical(it[d], 2)
            return carry

        lax.fori_loop(0, bpw // _L, quarter, 0)

        def chunk_body(ch, carry):
            cd = pl.ds(ch * chunk, chunk)
            cps = [
                pltpu.async_copy(g_ent.at[qh.at[cd]], bhe, sem),
                pltpu.async_copy(g_ent_im.at[qh.at[cd]], bhi, sem),
                pltpu.async_copy(g_rel.at[qr.at[cd]], brl, sem),
                pltpu.async_copy(g_ent.at[qt.at[cd]], bte, sem),
                pltpu.async_copy(g_ent_im.at[qt.at[cd]], bti, sem),
            ]
            for cp in cps:
                cp.wait()

            def grp(gq, c2):
                s16 = pl.ds(ch * chunk + gq * _L, _L)
                vh = ih[s16]
                vr = ir[s16]
                vt = it[s16]
                for lane in range(_L):
                    row = gq * _L + lane
                    oh = (vh[lane] & 3) * 32
                    orr = (vr[lane] & 3) * 32
                    ot = (vt[lane] & 3) * 32
                    acc = None
                    for half in range(2):
                        ho = half * _L
                        rl = brl[row, pl.ds(orr + ho, _L)]
                        sn, cs = _sincos(rl)
                        he = bhe[row, pl.ds(oh + ho, _L)]
                        hi = bhi[row, pl.ds(oh + ho, _L)]
                        te = bte[row, pl.ds(ot + ho, _L)]
                        ti = bti[row, pl.ds(ot + ho, _L)]
                        s_re = he * cs - hi * sn - te
                        s_im = he * sn + hi * cs - ti
                        m = _sqrt(s_re * s_re + s_im * s_im)
                        acc = m if acc is None else acc + m
                    plsc.store_scatter(
                        tr,
                        [iota * bpw + (ch * chunk + gq * _L + lane)],
                        acc)
                return c2

            lax.fori_loop(0, chunk // _L, grp, 0)
            return carry

        lax.fori_loop(0, n_chunks, chunk_body, 0)

        def rowsum(g, carry):
            acc = tr[pl.ds(g * _L, _L)]
            for k in range(1, _L):
                acc = acc + tr[pl.ds(k * bpw + g * _L, _L)]
            sc[pl.ds(g * _L, _L)] = acc - jnp.float32(_GAMMA)
            return carry

        lax.fori_loop(0, bpw // _L, rowsum, 0)
        pltpu.sync_copy(sc, out.at[pl.ds(base, bpw)])

    return score_kernel


def kernel(pos_sample, ent_embd, ent_embd_im, rel_embd):
    batch = pos_sample.shape[0]
    v = ent_embd.shape[0]
    ncols = v // 128
    tail = v - ncols * 128
    g_rows = (v + 3) // 4
    n_workers = 32
    bpw = batch // n_workers

    e_t = ent_embd.T
    ei_t = ent_embd_im.T
    r_t = rel_embd.T
    tails = jnp.concatenate([
        jnp.pad(t[ncols * 128:], ((0, 64 - tail), (0, 128 - t.shape[1])))
        for t in (ent_embd, ent_embd_im, rel_embd)
    ], axis=0)

    relayout = _make_relayout_kernel(v, ncols, tail, g_rows)
    g_ent, g_ent_im, g_rel = relayout(e_t, ei_t, r_t, tails)

    score_k = _make_score_kernel(batch, g_rows, bpw, 128)
    score = score_k(pos_sample[:, 0], pos_sample[:, 1], pos_sample[:, 2],
                    g_ent, g_ent_im, g_rel)
    return score.reshape(batch, 1)
